# baseline (speedup 1.0000x reference)
import functools

import jax
import jax.numpy as jnp
from jax import lax
from jax.experimental import pallas as pl
from jax.experimental.pallas import tpu as pltpu
from jax.experimental.pallas import tpu_sc as plsc

NC = 2
NS = 16
NW = NC * NS
K = 64
KD = 40

F32 = jnp.float32


def _sc_mesh():
    return plsc.VectorSubcoreMesh(
        core_axis_name="c", subcore_axis_name="s",
        num_cores=NC, num_subcores=NS)


def _deg_call(C, NP, ZR):
    @functools.partial(
        pl.kernel,
        out_type=jax.ShapeDtypeStruct((NC, NP, 16), F32),
        mesh=_sc_mesh(),
        scratch_types=[
            pltpu.VMEM((C, KD), jnp.int32),
            pltpu.VMEM((KD, 16), F32),
            pltpu.VMEM_SHARED((NP, 16), F32),
            pltpu.SemaphoreType.DMA,
        ],
        compiler_params=pltpu.CompilerParams(use_tc_tiling_on_sc=False),
    )
    def deg_kernel(dst_hbm, ones_hbm, z_hbm, out_hbm, dst_v, ones_v, acc, sem):
        cid = lax.axis_index("c")
        sid = lax.axis_index("s")
        wid = cid * NS + sid
        pltpu.sync_copy(z_hbm, acc.at[pl.ds(sid * ZR, ZR)])
        pltpu.sync_copy(dst_hbm.at[wid], dst_v)
        pltpu.sync_copy(ones_hbm, ones_v)
        plsc.subcore_barrier()

        def body(j, carry):
            pltpu.sync_copy(ones_v, acc.at[dst_v.at[j]], add=True)
            return carry

        lax.fori_loop(0, C, body, 0)
        plsc.subcore_barrier()
        pltpu.sync_copy(acc.at[pl.ds(sid * ZR, ZR)],
                        out_hbm.at[cid, pl.ds(sid * ZR, ZR)])

    return deg_kernel


def _scatter_call(D, C, NP, ZR):
    @functools.partial(
        pl.kernel,
        out_type=jax.ShapeDtypeStruct((NC, NP, D), F32),
        mesh=_sc_mesh(),
        scratch_types=[
            pltpu.VMEM((C, K), jnp.int32),
            pltpu.VMEM((C, K), jnp.int32),
            pltpu.VMEM((K, D), F32),
            pltpu.VMEM((K, D), F32),
            pltpu.VMEM_SHARED((NP, D), F32),
            pltpu.SemaphoreType.DMA,
            pltpu.SemaphoreType.DMA,
            pltpu.SemaphoreType.DMA,
            pltpu.SemaphoreType.DMA,
        ],
        compiler_params=pltpu.CompilerParams(use_tc_tiling_on_sc=False),
    )
    def scat_kernel(src_hbm, dst_hbm, z_hbm, g_hbm, out_hbm,
                    src_v, dst_v, rows0, rows1, acc, gs0, gs1, ss0, ss1):
        cid = lax.axis_index("c")
        sid = lax.axis_index("s")
        wid = cid * NS + sid
        pltpu.sync_copy(z_hbm, acc.at[pl.ds(sid * ZR, ZR)])
        pltpu.sync_copy(src_hbm.at[wid], src_v)
        pltpu.sync_copy(dst_hbm.at[wid], dst_v)
        plsc.subcore_barrier()

        pltpu.async_copy(g_hbm.at[src_v.at[0]], rows0, gs0)
        pltpu.async_copy(g_hbm.at[src_v.at[1]], rows1, gs1)

        def body(p, carry):
            j = 2 * p
            pltpu.make_async_copy(g_hbm.at[src_v.at[j]], rows0, gs0).wait()
            pltpu.async_copy(rows0, acc.at[dst_v.at[j]], ss0, add=True)
            pltpu.make_async_copy(
                g_hbm.at[src_v.at[j + 1]], rows1, gs1).wait()
            pltpu.async_copy(rows1, acc.at[dst_v.at[j + 1]], ss1, add=True)
            pltpu.make_async_copy(rows0, acc.at[dst_v.at[j]], ss0).wait()

            @pl.when(j + 2 < C)
            def _():
                pltpu.async_copy(g_hbm.at[src_v.at[j + 2]], rows0, gs0)

            pltpu.make_async_copy(rows1, acc.at[dst_v.at[j + 1]], ss1).wait()

            @pl.when(j + 3 < C)
            def _():
                pltpu.async_copy(g_hbm.at[src_v.at[j + 3]], rows1, gs1)

            return carry

        lax.fori_loop(0, C // 2, body, 0)
        if C % 2:
            pltpu.make_async_copy(
                g_hbm.at[src_v.at[C - 1]], rows0, gs0).wait()
            pltpu.sync_copy(rows0, acc.at[dst_v.at[C - 1]], add=True)
        plsc.subcore_barrier()
        pltpu.sync_copy(acc.at[pl.ds(sid * ZR, ZR)],
                        out_hbm.at[cid, pl.ds(sid * ZR, ZR)])

    return scat_kernel


def _tc1(x, W, degp, R):
    N, D = x.shape

    def body(x_ref, w_ref, dp_ref, g_ref, dinv_ref):
        d = dp_ref[0] + dp_ref[1] + 1.0
        dinv = lax.rsqrt(d)
        dinv_ref[...] = dinv
        h = jnp.dot(x_ref[...], w_ref[...], preferred_element_type=F32)
        g_ref[...] = h * dinv[:, :1]

    return pl.pallas_call(
        body,
        grid=(N // R,),
        in_specs=[
            pl.BlockSpec((R, D), lambda i: (i, 0)),
            pl.BlockSpec((D, D), lambda i: (0, 0)),
            pl.BlockSpec((NC, R, 16), lambda i: (0, i, 0)),
        ],
        out_specs=[
            pl.BlockSpec((R, D), lambda i: (i, 0)),
            pl.BlockSpec((R, 16), lambda i: (i, 0)),
        ],
        out_shape=[
            jax.ShapeDtypeStruct((N, D), F32),
            jax.ShapeDtypeStruct((N, 16), F32),
        ],
    )(x, W, degp)


def _tc2(agg, g, dinv16, b, W, R):
    N, D = g.shape

    def body(a_ref, g_ref, dinv_ref, b_ref, w_ref, o_ref):
        t = a_ref[0] + a_ref[1] + g_ref[...]
        dinv = dinv_ref[...][:, :1]
        h = jnp.maximum(t * dinv + b_ref[...], 0.0)
        o_ref[...] = jnp.dot(h, w_ref[...], preferred_element_type=F32) * dinv

    return pl.pallas_call(
        body,
        grid=(N // R,),
        in_specs=[
            pl.BlockSpec((NC, R, D), lambda i: (0, i, 0)),
            pl.BlockSpec((R, D), lambda i: (i, 0)),
            pl.BlockSpec((R, 16), lambda i: (i, 0)),
            pl.BlockSpec((1, D), lambda i: (0, 0)),
            pl.BlockSpec((D, D), lambda i: (0, 0)),
        ],
        out_specs=pl.BlockSpec((R, D), lambda i: (i, 0)),
        out_shape=jax.ShapeDtypeStruct((N, D), F32),
    )(agg, g, dinv16, b, W)


def _tc3(agg, g, dinv16, b, R):
    N, D = g.shape

    def body(a_ref, g_ref, dinv_ref, b_ref, o_ref):
        t = a_ref[0] + a_ref[1] + g_ref[...]
        dinv = dinv_ref[...][:, :1]
        o_ref[...] = t * dinv + b_ref[...]

    return pl.pallas_call(
        body,
        grid=(N // R,),
        in_specs=[
            pl.BlockSpec((NC, R, D), lambda i: (0, i, 0)),
            pl.BlockSpec((R, D), lambda i: (i, 0)),
            pl.BlockSpec((R, 16), lambda i: (i, 0)),
            pl.BlockSpec((1, D), lambda i: (0, 0)),
        ],
        out_specs=pl.BlockSpec((R, D), lambda i: (i, 0)),
        out_shape=jax.ShapeDtypeStruct((N, D), F32),
    )(agg, g, dinv16, b)


def kernel(x, edge_index, W1, b1, W2, b2):
    N, D = x.shape
    E = edge_index.shape[1]
    CD = E // (NW * KD)
    assert CD * NW * KD == E and N % NS == 0
    ZR = -(-N // NS)
    ZR += (-ZR) % 8
    NP = ZR * NS
    CS = -(-E // (NW * K))
    EP = CS * NW * K

    ei = edge_index.astype(jnp.int32)
    dst3 = ei[1].reshape(NW, CD, KD)
    pad = N + jnp.arange(EP - E, dtype=jnp.int32) % jnp.int32(NP - N)
    src3 = jnp.concatenate([ei[0], jnp.zeros((EP - E,), jnp.int32)])
    src3 = src3.reshape(NW, CS, K)
    dsc3 = jnp.concatenate([ei[1], pad]).reshape(NW, CS, K)
    ones16 = jnp.ones((KD, 16), F32)
    z16 = jnp.zeros((ZR, 16), F32)
    zD = jnp.zeros((ZR, D), F32)

    R = 1000
    degp = _deg_call(CD, NP, ZR)(dst3, ones16, z16)
    g1, dinv16 = _tc1(x, W1, degp, R)
    scat = _scatter_call(D, CS, NP, ZR)
    agg1 = scat(src3, dsc3, zD, g1)
    g2 = _tc2(agg1, g1, dinv16, b1.reshape(1, -1), W2, R)
    agg2 = scat(src3, dsc3, zD, g2)
    return _tc3(agg2, g2, dinv16, b2.reshape(1, -1), R)


# speedup vs baseline: 1.3551x; 1.3551x over previous
import functools

import jax
import jax.numpy as jnp
from jax import lax
from jax.experimental import pallas as pl
from jax.experimental.pallas import tpu as pltpu
from jax.experimental.pallas import tpu_sc as plsc

NC = 2
NS = 16
NW = NC * NS
K = 80
KD = 40

F32 = jnp.float32


def _sc_mesh():
    return plsc.VectorSubcoreMesh(
        core_axis_name="c", subcore_axis_name="s",
        num_cores=NC, num_subcores=NS)


def _deg_call(C, NP, ZR):
    @functools.partial(
        pl.kernel,
        out_type=jax.ShapeDtypeStruct((NC, NP, 16), F32),
        mesh=_sc_mesh(),
        scratch_types=[
            pltpu.VMEM((C, KD), jnp.int32),
            pltpu.VMEM((KD, 16), F32),
            pltpu.VMEM_SHARED((NP, 16), F32),
            pltpu.SemaphoreType.DMA,
        ],
        compiler_params=pltpu.CompilerParams(use_tc_tiling_on_sc=False),
    )
    def deg_kernel(dst_hbm, ones_hbm, z_hbm, out_hbm, dst_v, ones_v, acc, sem):
        cid = lax.axis_index("c")
        sid = lax.axis_index("s")
        wid = cid * NS + sid
        pltpu.sync_copy(z_hbm, acc.at[pl.ds(sid * ZR, ZR)])
        pltpu.sync_copy(dst_hbm.at[wid], dst_v)
        pltpu.sync_copy(ones_hbm, ones_v)
        plsc.subcore_barrier()

        def body(j, carry):
            pltpu.sync_copy(ones_v, acc.at[dst_v.at[j]], add=True)
            return carry

        lax.fori_loop(0, C, body, 0)
        plsc.subcore_barrier()
        pltpu.sync_copy(acc.at[pl.ds(sid * ZR, ZR)],
                        out_hbm.at[cid, pl.ds(sid * ZR, ZR)])

    return deg_kernel


def _scatter_call(D, C, NP, ZR):
    @functools.partial(
        pl.kernel,
        out_type=jax.ShapeDtypeStruct((NC, NP, D), F32),
        mesh=_sc_mesh(),
        scratch_types=[
            pltpu.VMEM((C, K), jnp.int32),
            pltpu.VMEM((C, K), jnp.int32),
            pltpu.VMEM((K, D), F32),
            pltpu.VMEM((K, D), F32),
            pltpu.VMEM_SHARED((NP, D), F32),
            pltpu.SemaphoreType.DMA,
            pltpu.SemaphoreType.DMA,
            pltpu.SemaphoreType.DMA,
            pltpu.SemaphoreType.DMA,
        ],
        compiler_params=pltpu.CompilerParams(use_tc_tiling_on_sc=False),
    )
    def scat_kernel(src_hbm, dst_hbm, z_hbm, g_hbm, out_hbm,
                    src_v, dst_v, rows0, rows1, acc, gs0, gs1, ss0, ss1):
        cid = lax.axis_index("c")
        sid = lax.axis_index("s")
        wid = cid * NS + sid
        pltpu.sync_copy(z_hbm, acc.at[pl.ds(sid * ZR, ZR)])
        pltpu.sync_copy(src_hbm.at[wid], src_v)
        pltpu.sync_copy(dst_hbm.at[wid], dst_v)
        plsc.subcore_barrier()

        pltpu.async_copy(g_hbm.at[src_v.at[0]], rows0, gs0)
        pltpu.async_copy(g_hbm.at[src_v.at[1]], rows1, gs1)

        def body(p, carry):
            j = 2 * p
            pltpu.make_async_copy(g_hbm.at[src_v.at[j]], rows0, gs0).wait()
            pltpu.async_copy(rows0, acc.at[dst_v.at[j]], ss0, add=True)
            pltpu.make_async_copy(
                g_hbm.at[src_v.at[j + 1]], rows1, gs1).wait()
            pltpu.async_copy(rows1, acc.at[dst_v.at[j + 1]], ss1, add=True)
            pltpu.make_async_copy(rows0, acc.at[dst_v.at[j]], ss0).wait()

            @pl.when(j + 2 < C)
            def _():
                pltpu.async_copy(g_hbm.at[src_v.at[j + 2]], rows0, gs0)

            pltpu.make_async_copy(rows1, acc.at[dst_v.at[j + 1]], ss1).wait()

            @pl.when(j + 3 < C)
            def _():
                pltpu.async_copy(g_hbm.at[src_v.at[j + 3]], rows1, gs1)

            return carry

        lax.fori_loop(0, C // 2, body, 0)
        if C % 2:
            pltpu.make_async_copy(
                g_hbm.at[src_v.at[C - 1]], rows0, gs0).wait()
            pltpu.sync_copy(rows0, acc.at[dst_v.at[C - 1]], add=True)
        plsc.subcore_barrier()
        pltpu.sync_copy(acc.at[pl.ds(sid * ZR, ZR)],
                        out_hbm.at[cid, pl.ds(sid * ZR, ZR)])

    return scat_kernel


def _tc1(x, W, degp, R):
    N, D = x.shape

    def body(x_ref, w_ref, dp_ref, g_ref, dinv_ref):
        d = dp_ref[0] + dp_ref[1] + 1.0
        dinv = lax.rsqrt(d)
        dinv_ref[...] = dinv
        h = jnp.dot(x_ref[...], w_ref[...], preferred_element_type=F32)
        g_ref[...] = h * dinv[:, :1]

    return pl.pallas_call(
        body,
        grid=(N // R,),
        in_specs=[
            pl.BlockSpec((R, D), lambda i: (i, 0)),
            pl.BlockSpec((D, D), lambda i: (0, 0)),
            pl.BlockSpec((NC, R, 16), lambda i: (0, i, 0)),
        ],
        out_specs=[
            pl.BlockSpec((R, D), lambda i: (i, 0)),
            pl.BlockSpec((R, 16), lambda i: (i, 0)),
        ],
        out_shape=[
            jax.ShapeDtypeStruct((N, D), F32),
            jax.ShapeDtypeStruct((N, 16), F32),
        ],
    )(x, W, degp)


def _tc2(agg, g, dinv16, b, W, R):
    N, D = g.shape

    def body(a_ref, g_ref, dinv_ref, b_ref, w_ref, o_ref):
        t = a_ref[0] + a_ref[1] + g_ref[...]
        dinv = dinv_ref[...][:, :1]
        h = jnp.maximum(t * dinv + b_ref[...], 0.0)
        o_ref[...] = jnp.dot(h, w_ref[...], preferred_element_type=F32) * dinv

    return pl.pallas_call(
        body,
        grid=(N // R,),
        in_specs=[
            pl.BlockSpec((NC, R, D), lambda i: (0, i, 0)),
            pl.BlockSpec((R, D), lambda i: (i, 0)),
            pl.BlockSpec((R, 16), lambda i: (i, 0)),
            pl.BlockSpec((1, D), lambda i: (0, 0)),
            pl.BlockSpec((D, D), lambda i: (0, 0)),
        ],
        out_specs=pl.BlockSpec((R, D), lambda i: (i, 0)),
        out_shape=jax.ShapeDtypeStruct((N, D), F32),
    )(agg, g, dinv16, b, W)


def _tc3(agg, g, dinv16, b, R):
    N, D = g.shape

    def body(a_ref, g_ref, dinv_ref, b_ref, o_ref):
        t = a_ref[0] + a_ref[1] + g_ref[...]
        dinv = dinv_ref[...][:, :1]
        o_ref[...] = t * dinv + b_ref[...]

    return pl.pallas_call(
        body,
        grid=(N // R,),
        in_specs=[
            pl.BlockSpec((NC, R, D), lambda i: (0, i, 0)),
            pl.BlockSpec((R, D), lambda i: (i, 0)),
            pl.BlockSpec((R, 16), lambda i: (i, 0)),
            pl.BlockSpec((1, D), lambda i: (0, 0)),
        ],
        out_specs=pl.BlockSpec((R, D), lambda i: (i, 0)),
        out_shape=jax.ShapeDtypeStruct((N, D), F32),
    )(agg, g, dinv16, b)


def kernel(x, edge_index, W1, b1, W2, b2):
    N, D = x.shape
    E = edge_index.shape[1]
    CD = E // (NW * KD)
    assert CD * NW * KD == E and N % NS == 0
    ZR = -(-N // NS)
    ZR += (-ZR) % 8
    NP = ZR * NS
    CS = -(-E // (NW * K))
    EP = CS * NW * K

    ei = edge_index.astype(jnp.int32)
    dst3 = ei[1].reshape(NW, CD, KD)
    pad = N + jnp.arange(EP - E, dtype=jnp.int32) % jnp.int32(NP - N)
    src3 = jnp.concatenate([ei[0], jnp.zeros((EP - E,), jnp.int32)])
    src3 = src3.reshape(NW, CS, K)
    dsc3 = jnp.concatenate([ei[1], pad]).reshape(NW, CS, K)
    ones16 = jnp.ones((KD, 16), F32)
    z16 = jnp.zeros((ZR, 16), F32)
    zD = jnp.zeros((ZR, D), F32)

    R = 1000
    degp = _deg_call(CD, NP, ZR)(dst3, ones16, z16)
    g1, dinv16 = _tc1(x, W1, degp, R)
    scat = _scatter_call(D, CS, NP, ZR)
    agg1 = scat(src3, dsc3, zD, g1)
    g2 = _tc2(agg1, g1, dinv16, b1.reshape(1, -1), W2, R)
    agg2 = scat(src3, dsc3, zD, g2)
    return _tc3(agg2, g2, dinv16, b2.reshape(1, -1), R)


# speedup vs baseline: 1.5407x; 1.1370x over previous
import functools

import jax
import jax.numpy as jnp
from jax import lax
from jax.experimental import pallas as pl
from jax.experimental.pallas import tpu as pltpu
from jax.experimental.pallas import tpu_sc as plsc

NC = 2
NS = 16
NW = NC * NS
K = 80
KD = 40

F32 = jnp.float32


def _sc_mesh():
    return plsc.VectorSubcoreMesh(
        core_axis_name="c", subcore_axis_name="s",
        num_cores=NC, num_subcores=NS)


def _deg_call(C, NP, ZR):
    @functools.partial(
        pl.kernel,
        out_type=jax.ShapeDtypeStruct((NC, NP, 16), F32),
        mesh=_sc_mesh(),
        scratch_types=[
            pltpu.VMEM((C, KD), jnp.int32),
            pltpu.VMEM((KD, 16), F32),
            pltpu.VMEM_SHARED((NP, 16), F32),
            pltpu.SemaphoreType.DMA,
        ],
        compiler_params=pltpu.CompilerParams(use_tc_tiling_on_sc=False),
    )
    def deg_kernel(dst_hbm, ones_hbm, z_hbm, out_hbm, dst_v, ones_v, acc, sem):
        cid = lax.axis_index("c")
        sid = lax.axis_index("s")
        wid = cid * NS + sid
        pltpu.sync_copy(z_hbm, acc.at[pl.ds(sid * ZR, ZR)])
        pltpu.sync_copy(dst_hbm.at[wid], dst_v)
        pltpu.sync_copy(ones_hbm, ones_v)
        plsc.subcore_barrier()

        def body(j, carry):
            pltpu.sync_copy(ones_v, acc.at[dst_v.at[j]], add=True)
            return carry

        lax.fori_loop(0, C, body, 0)
        plsc.subcore_barrier()
        pltpu.sync_copy(acc.at[pl.ds(sid * ZR, ZR)],
                        out_hbm.at[cid, pl.ds(sid * ZR, ZR)])

    return deg_kernel


def _scatter_call(D, C, NP, ZR, dt):
    @functools.partial(
        pl.kernel,
        out_type=jax.ShapeDtypeStruct((NC, NP, D), dt),
        mesh=_sc_mesh(),
        scratch_types=[
            pltpu.VMEM((C, K), jnp.int32),
            pltpu.VMEM((C, K), jnp.int32),
            pltpu.VMEM((K, D), dt),
            pltpu.VMEM((K, D), dt),
            pltpu.VMEM_SHARED((NP, D), dt),
            pltpu.SemaphoreType.DMA,
            pltpu.SemaphoreType.DMA,
            pltpu.SemaphoreType.DMA,
            pltpu.SemaphoreType.DMA,
        ],
        compiler_params=pltpu.CompilerParams(use_tc_tiling_on_sc=False),
    )
    def scat_kernel(src_hbm, dst_hbm, z_hbm, g_hbm, out_hbm,
                    src_v, dst_v, rows0, rows1, acc, gs0, gs1, ss0, ss1):
        cid = lax.axis_index("c")
        sid = lax.axis_index("s")
        wid = cid * NS + sid
        pltpu.sync_copy(z_hbm, acc.at[pl.ds(sid * ZR, ZR)])
        pltpu.sync_copy(src_hbm.at[wid], src_v)
        pltpu.sync_copy(dst_hbm.at[wid], dst_v)
        plsc.subcore_barrier()

        pltpu.async_copy(g_hbm.at[src_v.at[0]], rows0, gs0)
        pltpu.async_copy(g_hbm.at[src_v.at[1]], rows1, gs1)

        def body(p, carry):
            j = 2 * p
            pltpu.make_async_copy(g_hbm.at[src_v.at[j]], rows0, gs0).wait()
            pltpu.async_copy(rows0, acc.at[dst_v.at[j]], ss0, add=True)
            pltpu.make_async_copy(
                g_hbm.at[src_v.at[j + 1]], rows1, gs1).wait()
            pltpu.async_copy(rows1, acc.at[dst_v.at[j + 1]], ss1, add=True)
            pltpu.make_async_copy(rows0, acc.at[dst_v.at[j]], ss0).wait()

            @pl.when(j + 2 < C)
            def _():
                pltpu.async_copy(g_hbm.at[src_v.at[j + 2]], rows0, gs0)

            pltpu.make_async_copy(rows1, acc.at[dst_v.at[j + 1]], ss1).wait()

            @pl.when(j + 3 < C)
            def _():
                pltpu.async_copy(g_hbm.at[src_v.at[j + 3]], rows1, gs1)

            return carry

        lax.fori_loop(0, C // 2, body, 0)
        if C % 2:
            pltpu.make_async_copy(
                g_hbm.at[src_v.at[C - 1]], rows0, gs0).wait()
            pltpu.sync_copy(rows0, acc.at[dst_v.at[C - 1]], add=True)
        plsc.subcore_barrier()
        pltpu.sync_copy(acc.at[pl.ds(sid * ZR, ZR)],
                        out_hbm.at[cid, pl.ds(sid * ZR, ZR)])

    return scat_kernel


def _tc1(x, W, degp, R):
    N, D = x.shape

    def body(x_ref, w_ref, dp_ref, g_ref, gb_ref, dinv_ref):
        d = dp_ref[0] + dp_ref[1] + 1.0
        dinv = lax.rsqrt(d)
        dinv_ref[...] = dinv
        h = jnp.dot(x_ref[...], w_ref[...], preferred_element_type=F32)
        g = h * dinv[:, :1]
        g_ref[...] = g
        gb_ref[...] = g.astype(jnp.bfloat16)

    return pl.pallas_call(
        body,
        grid=(N // R,),
        in_specs=[
            pl.BlockSpec((R, D), lambda i: (i, 0)),
            pl.BlockSpec((D, D), lambda i: (0, 0)),
            pl.BlockSpec((NC, R, 16), lambda i: (0, i, 0)),
        ],
        out_specs=[
            pl.BlockSpec((R, D), lambda i: (i, 0)),
            pl.BlockSpec((R, D), lambda i: (i, 0)),
            pl.BlockSpec((R, 16), lambda i: (i, 0)),
        ],
        out_shape=[
            jax.ShapeDtypeStruct((N, D), F32),
            jax.ShapeDtypeStruct((N, D), jnp.bfloat16),
            jax.ShapeDtypeStruct((N, 16), F32),
        ],
    )(x, W, degp)


def _tc2(agg, g, dinv16, b, W, R):
    N, D = g.shape

    def body(a_ref, g_ref, dinv_ref, b_ref, w_ref, o_ref, ob_ref):
        t = (a_ref[0] + a_ref[1]).astype(F32) + g_ref[...]
        dinv = dinv_ref[...][:, :1]
        h = jnp.maximum(t * dinv + b_ref[...], 0.0)
        o = jnp.dot(h, w_ref[...], preferred_element_type=F32) * dinv
        o_ref[...] = o
        ob_ref[...] = o.astype(jnp.bfloat16)

    return pl.pallas_call(
        body,
        grid=(N // R,),
        in_specs=[
            pl.BlockSpec((NC, R, D), lambda i: (0, i, 0)),
            pl.BlockSpec((R, D), lambda i: (i, 0)),
            pl.BlockSpec((R, 16), lambda i: (i, 0)),
            pl.BlockSpec((1, D), lambda i: (0, 0)),
            pl.BlockSpec((D, D), lambda i: (0, 0)),
        ],
        out_specs=[
            pl.BlockSpec((R, D), lambda i: (i, 0)),
            pl.BlockSpec((R, D), lambda i: (i, 0)),
        ],
        out_shape=[
            jax.ShapeDtypeStruct((N, D), F32),
            jax.ShapeDtypeStruct((N, D), jnp.bfloat16),
        ],
    )(agg, g, dinv16, b, W)


def _tc3(agg, g, dinv16, b, R):
    N, D = g.shape

    def body(a_ref, g_ref, dinv_ref, b_ref, o_ref):
        t = (a_ref[0] + a_ref[1]).astype(F32) + g_ref[...]
        dinv = dinv_ref[...][:, :1]
        o_ref[...] = t * dinv + b_ref[...]

    return pl.pallas_call(
        body,
        grid=(N // R,),
        in_specs=[
            pl.BlockSpec((NC, R, D), lambda i: (0, i, 0)),
            pl.BlockSpec((R, D), lambda i: (i, 0)),
            pl.BlockSpec((R, 16), lambda i: (i, 0)),
            pl.BlockSpec((1, D), lambda i: (0, 0)),
        ],
        out_specs=pl.BlockSpec((R, D), lambda i: (i, 0)),
        out_shape=jax.ShapeDtypeStruct((N, D), F32),
    )(agg, g, dinv16, b)


def kernel(x, edge_index, W1, b1, W2, b2):
    N, D = x.shape
    E = edge_index.shape[1]
    CD = E // (NW * KD)
    assert CD * NW * KD == E and N % NS == 0
    ZR = -(-N // NS)
    ZR += (-ZR) % 8
    NP = ZR * NS
    CS = -(-E // (NW * K))
    EP = CS * NW * K

    ei = edge_index.astype(jnp.int32)
    dst3 = ei[1].reshape(NW, CD, KD)
    pad = N + jnp.arange(EP - E, dtype=jnp.int32) % jnp.int32(NP - N)
    src3 = jnp.concatenate([ei[0], jnp.zeros((EP - E,), jnp.int32)])
    src3 = src3.reshape(NW, CS, K)
    dsc3 = jnp.concatenate([ei[1], pad]).reshape(NW, CS, K)
    ones16 = jnp.ones((KD, 16), F32)
    z16 = jnp.zeros((ZR, 16), F32)
    zD = jnp.zeros((ZR, D), jnp.bfloat16)

    R = 1000
    degp = _deg_call(CD, NP, ZR)(dst3, ones16, z16)
    g1, g1b, dinv16 = _tc1(x, W1, degp, R)
    scat = _scatter_call(D, CS, NP, ZR, jnp.bfloat16)
    agg1 = scat(src3, dsc3, zD, g1b)
    g2, g2b = _tc2(agg1, g1, dinv16, b1.reshape(1, -1), W2, R)
    agg2 = scat(src3, dsc3, zD, g2b)
    return _tc3(agg2, g2, dinv16, b2.reshape(1, -1), R)
